# baseline (speedup 1.0000x reference)
import functools

import jax
import jax.numpy as jnp
from jax import lax
from jax.experimental import pallas as pl
from jax.experimental.pallas import tpu as pltpu
from jax.experimental.pallas import tpu_sc as plsc

N = 50000
E = 800000
D_IN = 128
H = 64
HH = H // 2

BN = 784
N_PAD = 50176
E_PAD = 819200
ER = E_PAD // 64
ER_P = ER + 4
TILE_ROWS = N_PAD // 16

_MESH = plsc.VectorSubcoreMesh(core_axis_name="c", subcore_axis_name="s")
_SC_PARAMS = pltpu.CompilerParams(needs_layout_passes=False,
                                  use_tc_tiling_on_sc=False)


def _prep_body(x_ref, win_ref, bin_ref, w1_ref, b1_ref, wg0_ref,
               emb_ref, a_ref, b_ref, xw_ref):
    emb = jnp.dot(x_ref[...], win_ref[...],
                  preferred_element_type=jnp.float32) + bin_ref[...]
    emb_ref[...] = emb
    w1 = w1_ref[...]
    a_ref[...] = jnp.dot(emb, w1[:H], preferred_element_type=jnp.float32) \
        + b1_ref[...]
    b_ref[...] = jnp.dot(emb, w1[H:], preferred_element_type=jnp.float32)
    xw = jnp.dot(emb, wg0_ref[...], preferred_element_type=jnp.float32)
    xw_ref[0] = xw[:, :HH]
    xw_ref[1] = xw[:, HH:]


def _prep_call(x_pad, W_in, b_in, W1, b1, Wg0):
    mat64 = jax.ShapeDtypeStruct((N_PAD, H), jnp.float32)
    return pl.pallas_call(
        _prep_body,
        grid=(N_PAD // BN,),
        in_specs=[
            pl.BlockSpec((BN, D_IN), lambda i: (i, 0)),
            pl.BlockSpec((D_IN, H), lambda i: (0, 0)),
            pl.BlockSpec((1, H), lambda i: (0, 0)),
            pl.BlockSpec((2 * H, H), lambda i: (0, 0)),
            pl.BlockSpec((1, H), lambda i: (0, 0)),
            pl.BlockSpec((H, H), lambda i: (0, 0)),
        ],
        out_specs=[
            pl.BlockSpec((BN, H), lambda i: (i, 0)),
            pl.BlockSpec((BN, H), lambda i: (i, 0)),
            pl.BlockSpec((BN, H), lambda i: (i, 0)),
            pl.BlockSpec((2, BN, HH), lambda i: (0, i, 0)),
        ],
        out_shape=[mat64, mat64, mat64,
                   jax.ShapeDtypeStruct((2, N_PAD, HH), jnp.float32)],
    )(x_pad, W_in, b_in, W1, b1, Wg0)


def _mid_body(s_ref, bprev_ref, wg_ref, accin_ref, accout_ref, xw_ref):
    h = jnp.concatenate([s_ref[0], s_ref[1]], axis=1) + bprev_ref[...]
    h = jnp.maximum(h, 0.0)
    accout_ref[...] = accin_ref[...] + h
    xw = jnp.dot(h, wg_ref[...], preferred_element_type=jnp.float32)
    xw_ref[0] = xw[:, :HH]
    xw_ref[1] = xw[:, HH:]


def _mid_call(S, b_prev, Wg, acc_in):
    return pl.pallas_call(
        _mid_body,
        grid=(N_PAD // BN,),
        in_specs=[
            pl.BlockSpec((2, BN, HH), lambda i: (0, i, 0)),
            pl.BlockSpec((1, H), lambda i: (0, 0)),
            pl.BlockSpec((H, H), lambda i: (0, 0)),
            pl.BlockSpec((BN, H), lambda i: (i, 0)),
        ],
        out_specs=[
            pl.BlockSpec((BN, H), lambda i: (i, 0)),
            pl.BlockSpec((2, BN, HH), lambda i: (0, i, 0)),
        ],
        out_shape=[jax.ShapeDtypeStruct((N_PAD, H), jnp.float32),
                   jax.ShapeDtypeStruct((2, N_PAD, HH), jnp.float32)],
    )(S, b_prev, Wg, acc_in)


def _final_body(s_ref, blast_ref, accin_ref, out_ref):
    h = jnp.concatenate([s_ref[0], s_ref[1]], axis=1) + blast_ref[...]
    out_ref[...] = (accin_ref[...] + h) * 0.25


def _final_call(S, b_last, acc_in):
    return pl.pallas_call(
        _final_body,
        grid=(N_PAD // BN,),
        in_specs=[
            pl.BlockSpec((2, BN, HH), lambda i: (0, i, 0)),
            pl.BlockSpec((1, H), lambda i: (0, 0)),
            pl.BlockSpec((BN, H), lambda i: (i, 0)),
        ],
        out_specs=pl.BlockSpec((BN, H), lambda i: (i, 0)),
        out_shape=jax.ShapeDtypeStruct((N_PAD, H), jnp.float32),
    )(S, b_last, acc_in)


_MROWS = E_PAD // 32 // 64
_MCH = 4
_MCHUNKS = _MROWS // _MCH
_MB = _MCH * 64


@functools.partial(
    pl.kernel,
    out_type=jax.ShapeDtypeStruct((ER, 64), jnp.float32),
    mesh=_MESH,
    compiler_params=_SC_PARAMS,
    scratch_types=[
        pltpu.VMEM((_MCH, 64), jnp.int32),
        pltpu.VMEM((_MCH, 64), jnp.int32),
        pltpu.VMEM((_MB, H), jnp.float32),
        pltpu.VMEM((_MB, H), jnp.float32),
        pltpu.VMEM((_MCH, 64), jnp.int32),
        pltpu.VMEM((_MCH, 64), jnp.int32),
        pltpu.VMEM((_MB, H), jnp.float32),
        pltpu.VMEM((_MB, H), jnp.float32),
        pltpu.VMEM((_MCH, 64), jnp.float32),
        pltpu.VMEM((H,), jnp.float32),
        pltpu.VMEM((16,), jnp.float32),
        pltpu.SemaphoreType.DMA,
        pltpu.SemaphoreType.DMA,
    ],
)
def _mask_kernel(row_hbm, col_hbm, a_hbm, b_hbm, w2_hbm, b2_hbm, mask_hbm,
                 ridx0, cidx0, arows0, brows0, ridx1, cidx1, arows1, brows1,
                 mout, w2v, b2v, sem0, sem1):
    c = lax.axis_index("c")
    s = lax.axis_index("s")
    wid = s * 2 + c
    pltpu.sync_copy(w2_hbm, w2v)
    pltpu.sync_copy(b2_hbm, b2v)
    base = wid * _MROWS
    b2vec = b2v[...]
    w2c = [w2v[pl.ds(k * 16, 16)] for k in range(H // 16)]
    zvec = jnp.zeros((16,), jnp.float32)

    def _fire(ci, ridx, cidx, arows, brows, sem):
        rb = base + ci * _MCH
        pltpu.sync_copy(row_hbm.at[pl.ds(rb, _MCH)], ridx)
        pltpu.sync_copy(col_hbm.at[pl.ds(rb, _MCH)], cidx)
        for j in range(_MCH):
            pltpu.async_copy(
                a_hbm.at[ridx.at[j]], arows.at[pl.ds(j * 64, 64)], sem)
            pltpu.async_copy(
                b_hbm.at[cidx.at[j]], brows.at[pl.ds(j * 64, 64)], sem)

    def _wait(ridx, cidx, arows, brows, sem):
        for j in range(_MCH):
            pltpu.make_async_copy(
                a_hbm.at[ridx.at[j]], arows.at[pl.ds(j * 64, 64)], sem).wait()
            pltpu.make_async_copy(
                b_hbm.at[cidx.at[j]], brows.at[pl.ds(j * 64, 64)], sem).wait()

    def _process(ci, arows, brows):
        rb = base + ci * _MCH

        @plsc.parallel_loop(0, _MB // 16, 1, unroll=2)
        def _grp(g):
            rowi = g * 16 + lax.iota(jnp.int32, 16)
            accs = [b2vec, zvec, zvec, zvec]
            for d in range(H):
                di = jnp.full((16,), d, jnp.int32)
                av = plsc.load_gather(arows, [rowi, di])
                bv = plsc.load_gather(brows, [rowi, di])
                accs[d % 4] = accs[d % 4] \
                    + jnp.maximum(av + bv, 0.0) * w2c[d // 16][d % 16]
            acc = (accs[0] + accs[1]) + (accs[2] + accs[3])
            m = 1.0 / (1.0 + jnp.exp(-acc))
            mout[g // 4, pl.ds((g % 4) * 16, 16)] = m

        pltpu.sync_copy(mout, mask_hbm.at[pl.ds(rb, _MCH)])

    _fire(0, ridx0, cidx0, arows0, brows0, sem0)

    def body(ci, carry):
        _fire(2 * ci + 1, ridx1, cidx1, arows1, brows1, sem1)
        _wait(ridx0, cidx0, arows0, brows0, sem0)
        _process(2 * ci, arows0, brows0)
        _fire(2 * ci + 2, ridx0, cidx0, arows0, brows0, sem0)
        _wait(ridx1, cidx1, arows1, brows1, sem1)
        _process(2 * ci + 1, arows1, brows1)
        return carry

    lax.fori_loop(0, _MCHUNKS // 2, body, 0)
    _wait(ridx0, cidx0, arows0, brows0, sem0)


_GROWS = E_PAD // 16 // 64
_GCH = 4
_GCHUNKS = _GROWS // _GCH
_GB = _GCH * 64


@functools.partial(
    pl.kernel,
    out_type=jax.ShapeDtypeStruct((2 * N_PAD, HH), jnp.float32),
    mesh=_MESH,
    compiler_params=_SC_PARAMS,
    scratch_types=[
        pltpu.VMEM((_GCH, 64), jnp.int32),
        pltpu.VMEM((_GCH, 64), jnp.int32),
        pltpu.VMEM((_GCH, 64), jnp.float32),
        pltpu.VMEM((_GB, HH), jnp.float32),
        pltpu.VMEM((_GCH, 64), jnp.int32),
        pltpu.VMEM((_GCH, 64), jnp.int32),
        pltpu.VMEM((_GCH, 64), jnp.float32),
        pltpu.VMEM((_GB, HH), jnp.float32),
        pltpu.VMEM((_GB, HH), jnp.float32),
        pltpu.VMEM_SHARED((N_PAD, HH), jnp.float32),
        pltpu.SemaphoreType.DMA,
        pltpu.SemaphoreType.DMA,
        pltpu.SemaphoreType.DMA,
    ],
)
def _gsa_kernel(row_hbm, col_hbm, mask_hbm, xw_hbm, out_hbm,
                ridx0, cidx0, mch0, rows0, ridx1, cidx1, mch1, rows1,
                msg, accum, sem0, sem1, sem2):
    c = lax.axis_index("c")
    s = lax.axis_index("s")
    ebase = s * _GROWS
    off = c * N_PAD

    _CCHUNKS = [_GB] * (TILE_ROWS // _GB) + [TILE_ROWS % _GB]

    def _fire(ci, ridx, cidx, mch, rows, sem):
        rb = ebase + ci * _GCH
        pltpu.sync_copy(row_hbm.at[pl.ds(rb, _GCH)], ridx)
        pltpu.sync_copy(col_hbm.at[pl.ds(rb, _GCH)], cidx)
        pltpu.sync_copy(mask_hbm.at[pl.ds(rb, _GCH)], mch)
        for i in range(_GCH):
            for k in range(4):
                ridx[i, pl.ds(k * 16, 16)] = ridx[i, pl.ds(k * 16, 16)] + off
        for j in range(_GCH):
            pltpu.async_copy(xw_hbm.at[ridx.at[j]],
                             rows.at[pl.ds(j * 64, 64)], sem)

    def _wait_gathers(ridx, rows, sem):
        for j in range(_GCH):
            pltpu.make_async_copy(xw_hbm.at[ridx.at[j]],
                                  rows.at[pl.ds(j * 64, 64)], sem).wait()

    def _process(cidx, mch, rows):
        @plsc.parallel_loop(0, _GB // 16, 1, unroll=4)
        def _scale(g):
            m = mch[g // 4, pl.ds((g % 4) * 16, 16)]
            rowi = g * 16 + lax.iota(jnp.int32, 16)
            for dd in range(HH):
                di = jnp.full((16,), dd, jnp.int32)
                v = plsc.load_gather(rows, [rowi, di])
                plsc.store_scatter(msg, [rowi, di], v * m)

        descs = [pltpu.async_copy(msg.at[pl.ds(j * 64, 64)],
                                  accum.at[cidx.at[j]], sem2, add=True)
                 for j in range(_GCH)]
        for d in descs:
            d.wait()

    _fire(0, ridx0, cidx0, mch0, rows0, sem0)

    def zrow(i, carry):
        msg[i, pl.ds(0, 16)] = jnp.zeros((16,), jnp.float32)
        msg[i, pl.ds(16, 16)] = jnp.zeros((16,), jnp.float32)
        return carry

    lax.fori_loop(0, _GB, zrow, 0)
    zbase = s * TILE_ROWS
    for sz in _CCHUNKS:
        pltpu.sync_copy(msg.at[pl.ds(0, sz)], accum.at[pl.ds(zbase, sz)])
        zbase = zbase + sz
    plsc.subcore_barrier()

    def body(ci, carry):
        _fire(2 * ci + 1, ridx1, cidx1, mch1, rows1, sem1)
        _wait_gathers(ridx0, rows0, sem0)
        _process(cidx0, mch0, rows0)
        _fire(2 * ci + 2, ridx0, cidx0, mch0, rows0, sem0)
        _wait_gathers(ridx1, rows1, sem1)
        _process(cidx1, mch1, rows1)
        return carry

    lax.fori_loop(0, _GCHUNKS // 2, body, 0)
    _wait_gathers(ridx0, rows0, sem0)
    plsc.subcore_barrier()

    rb = s * TILE_ROWS
    for sz in _CCHUNKS:
        pltpu.sync_copy(accum.at[pl.ds(rb, sz)], msg.at[pl.ds(0, sz)])
        pltpu.sync_copy(msg.at[pl.ds(0, sz)], out_hbm.at[pl.ds(off + rb, sz)])
        rb = rb + sz


def kernel(x, edge_index, W_in, b_in, W1, b1, W2, b2,
           Wg0, bg0, Wg1, bg1, Wg2, bg2):
    x_pad = jnp.pad(x, ((0, N_PAD - N), (0, 0)))
    row = edge_index[0]
    col = edge_index[1]
    npad = ER_P * 64 - E
    spread = (jnp.arange(npad, dtype=jnp.int32) * 61) % N
    row2d = jnp.concatenate([row, spread]).reshape(ER_P, 64)
    col2d = jnp.concatenate([col, spread]).reshape(ER_P, 64)

    emb, A, B, xw0 = _prep_call(x_pad, W_in, b_in.reshape(1, H),
                                W1, b1.reshape(1, H), Wg0)
    mask2d = _mask_kernel(row2d, col2d, A, B, W2.reshape(H),
                          jnp.broadcast_to(b2, (16,)))
    mask2d = jnp.concatenate(
        [mask2d[:E // 64], jnp.zeros((ER_P - E // 64, 64), jnp.float32)])

    S1 = _gsa_kernel(row2d, col2d, mask2d, xw0.reshape(2 * N_PAD, HH))
    acc1, xw1 = _mid_call(S1.reshape(2, N_PAD, HH), bg0.reshape(1, H),
                          Wg1, emb)
    S2 = _gsa_kernel(row2d, col2d, mask2d, xw1.reshape(2 * N_PAD, HH))
    acc2, xw2 = _mid_call(S2.reshape(2, N_PAD, HH), bg1.reshape(1, H),
                          Wg2, acc1)
    S3 = _gsa_kernel(row2d, col2d, mask2d, xw2.reshape(2 * N_PAD, HH))
    out = _final_call(S3.reshape(2, N_PAD, HH), bg2.reshape(1, H), acc2)
    return out[:N]


# speedup vs baseline: 2.5762x; 2.5762x over previous
import functools

import jax
import jax.numpy as jnp
from jax import lax
from jax.experimental import pallas as pl
from jax.experimental.pallas import tpu as pltpu
from jax.experimental.pallas import tpu_sc as plsc

N = 50000
E = 800000
D_IN = 128
H = 64
HH = H // 2

BN = 784
N_PAD = 50176
E_PAD = 819200
ER = E_PAD // 64
ER_P = ER + 4
TILE_ROWS = N_PAD // 16

_MESH = plsc.VectorSubcoreMesh(core_axis_name="c", subcore_axis_name="s")
_SC_PARAMS = pltpu.CompilerParams(needs_layout_passes=False,
                                  use_tc_tiling_on_sc=False)


def _prep_body(x_ref, win_ref, bin_ref, w1_ref, b1_ref, wg0_ref,
               emb_ref, a_ref, b_ref, xw_ref):
    emb = jnp.dot(x_ref[...], win_ref[...],
                  preferred_element_type=jnp.float32) + bin_ref[...]
    emb_ref[...] = emb
    w1 = w1_ref[...]
    a_ref[...] = jnp.dot(emb, w1[:H], preferred_element_type=jnp.float32) \
        + b1_ref[...]
    b_ref[...] = jnp.dot(emb, w1[H:], preferred_element_type=jnp.float32)
    xw = jnp.dot(emb, wg0_ref[...], preferred_element_type=jnp.float32)
    xw_ref[0] = xw[:, :HH]
    xw_ref[1] = xw[:, HH:]


def _prep_call(x_pad, W_in, b_in, W1, b1, Wg0):
    mat64 = jax.ShapeDtypeStruct((N_PAD, H), jnp.float32)
    return pl.pallas_call(
        _prep_body,
        grid=(N_PAD // BN,),
        in_specs=[
            pl.BlockSpec((BN, D_IN), lambda i: (i, 0)),
            pl.BlockSpec((D_IN, H), lambda i: (0, 0)),
            pl.BlockSpec((1, H), lambda i: (0, 0)),
            pl.BlockSpec((2 * H, H), lambda i: (0, 0)),
            pl.BlockSpec((1, H), lambda i: (0, 0)),
            pl.BlockSpec((H, H), lambda i: (0, 0)),
        ],
        out_specs=[
            pl.BlockSpec((BN, H), lambda i: (i, 0)),
            pl.BlockSpec((BN, H), lambda i: (i, 0)),
            pl.BlockSpec((BN, H), lambda i: (i, 0)),
            pl.BlockSpec((2, BN, HH), lambda i: (0, i, 0)),
        ],
        out_shape=[mat64, mat64, mat64,
                   jax.ShapeDtypeStruct((2, N_PAD, HH), jnp.float32)],
    )(x_pad, W_in, b_in, W1, b1, Wg0)


def _mid_body(s_ref, bprev_ref, wg_ref, accin_ref, accout_ref, xw_ref):
    h = jnp.concatenate([s_ref[0], s_ref[1]], axis=1) + bprev_ref[...]
    h = jnp.maximum(h, 0.0)
    accout_ref[...] = accin_ref[...] + h
    xw = jnp.dot(h, wg_ref[...], preferred_element_type=jnp.float32)
    xw_ref[0] = xw[:, :HH]
    xw_ref[1] = xw[:, HH:]


def _mid_call(S, b_prev, Wg, acc_in):
    return pl.pallas_call(
        _mid_body,
        grid=(N_PAD // BN,),
        in_specs=[
            pl.BlockSpec((2, BN, HH), lambda i: (0, i, 0)),
            pl.BlockSpec((1, H), lambda i: (0, 0)),
            pl.BlockSpec((H, H), lambda i: (0, 0)),
            pl.BlockSpec((BN, H), lambda i: (i, 0)),
        ],
        out_specs=[
            pl.BlockSpec((BN, H), lambda i: (i, 0)),
            pl.BlockSpec((2, BN, HH), lambda i: (0, i, 0)),
        ],
        out_shape=[jax.ShapeDtypeStruct((N_PAD, H), jnp.float32),
                   jax.ShapeDtypeStruct((2, N_PAD, HH), jnp.float32)],
    )(S, b_prev, Wg, acc_in)


def _final_body(s_ref, blast_ref, accin_ref, out_ref):
    h = jnp.concatenate([s_ref[0], s_ref[1]], axis=1) + blast_ref[...]
    out_ref[...] = (accin_ref[...] + h) * 0.25


def _final_call(S, b_last, acc_in):
    return pl.pallas_call(
        _final_body,
        grid=(N_PAD // BN,),
        in_specs=[
            pl.BlockSpec((2, BN, HH), lambda i: (0, i, 0)),
            pl.BlockSpec((1, H), lambda i: (0, 0)),
            pl.BlockSpec((BN, H), lambda i: (i, 0)),
        ],
        out_specs=pl.BlockSpec((BN, H), lambda i: (i, 0)),
        out_shape=jax.ShapeDtypeStruct((N_PAD, H), jnp.float32),
    )(S, b_last, acc_in)


_MROWS = E_PAD // 32 // 64
_MCH = 4
_MCHUNKS = _MROWS // _MCH
_MB = _MCH * 64


@functools.partial(
    pl.kernel,
    out_type=jax.ShapeDtypeStruct((ER, 64), jnp.float32),
    mesh=_MESH,
    compiler_params=_SC_PARAMS,
    scratch_types=[
        pltpu.VMEM((_MCH, 64), jnp.int32),
        pltpu.VMEM((_MCH, 64), jnp.int32),
        pltpu.VMEM((_MB, H), jnp.float32),
        pltpu.VMEM((_MB, H), jnp.float32),
        pltpu.VMEM((_MCH, 64), jnp.int32),
        pltpu.VMEM((_MCH, 64), jnp.int32),
        pltpu.VMEM((_MB, H), jnp.float32),
        pltpu.VMEM((_MB, H), jnp.float32),
        pltpu.VMEM((_MCH, 64), jnp.float32),
        pltpu.VMEM((H,), jnp.float32),
        pltpu.VMEM((16,), jnp.float32),
        pltpu.SemaphoreType.DMA,
        pltpu.SemaphoreType.DMA,
    ],
)
def _mask_kernel(row_hbm, col_hbm, a_hbm, b_hbm, w2_hbm, b2_hbm, mask_hbm,
                 ridx0, cidx0, arows0, brows0, ridx1, cidx1, arows1, brows1,
                 mout, w2v, b2v, sem0, sem1):
    c = lax.axis_index("c")
    s = lax.axis_index("s")
    wid = s * 2 + c
    pltpu.sync_copy(w2_hbm, w2v)
    pltpu.sync_copy(b2_hbm, b2v)
    base = wid * _MROWS
    b2vec = b2v[...]
    w2c = [w2v[pl.ds(k * 16, 16)] for k in range(H // 16)]
    zvec = jnp.zeros((16,), jnp.float32)

    def _fire(ci, ridx, cidx, arows, brows, sem):
        rb = base + ci * _MCH
        pltpu.sync_copy(row_hbm.at[pl.ds(rb, _MCH)], ridx)
        pltpu.sync_copy(col_hbm.at[pl.ds(rb, _MCH)], cidx)
        for j in range(_MCH):
            pltpu.async_copy(
                a_hbm.at[ridx.at[j]], arows.at[pl.ds(j * 64, 64)], sem)
            pltpu.async_copy(
                b_hbm.at[cidx.at[j]], brows.at[pl.ds(j * 64, 64)], sem)

    def _wait(ridx, cidx, arows, brows, sem):
        for j in range(_MCH):
            pltpu.make_async_copy(
                a_hbm.at[ridx.at[j]], arows.at[pl.ds(j * 64, 64)], sem).wait()
            pltpu.make_async_copy(
                b_hbm.at[cidx.at[j]], brows.at[pl.ds(j * 64, 64)], sem).wait()

    lane = lax.iota(jnp.int32, 16)

    def _process(ci, arows, brows):
        rb = base + ci * _MCH

        @plsc.parallel_loop(0, _MB // 16, 1)
        def _grp(g):
            macc = zvec
            for j in range(16):
                e = g * 16 + j
                p = zvec
                for k in range(H // 16):
                    av = arows[e, pl.ds(k * 16, 16)]
                    bv = brows[e, pl.ds(k * 16, 16)]
                    p = p + jnp.maximum(av + bv, 0.0) * w2c[k]
                sj = jnp.sum(p)
                macc = jnp.where(lane == j, zvec + sj, macc)
            m = 1.0 / (1.0 + jnp.exp(-(macc + b2vec)))
            mout[g // 4, pl.ds((g % 4) * 16, 16)] = m

        pltpu.sync_copy(mout, mask_hbm.at[pl.ds(rb, _MCH)])

    _fire(0, ridx0, cidx0, arows0, brows0, sem0)

    def body(ci, carry):
        _fire(2 * ci + 1, ridx1, cidx1, arows1, brows1, sem1)
        _wait(ridx0, cidx0, arows0, brows0, sem0)
        _process(2 * ci, arows0, brows0)
        _fire(2 * ci + 2, ridx0, cidx0, arows0, brows0, sem0)
        _wait(ridx1, cidx1, arows1, brows1, sem1)
        _process(2 * ci + 1, arows1, brows1)
        return carry

    lax.fori_loop(0, _MCHUNKS // 2, body, 0)
    _wait(ridx0, cidx0, arows0, brows0, sem0)


_GROWS = E_PAD // 16 // 64
_GCH = 4
_GCHUNKS = _GROWS // _GCH
_GB = _GCH * 64


@functools.partial(
    pl.kernel,
    out_type=jax.ShapeDtypeStruct((2 * N_PAD, HH), jnp.float32),
    mesh=_MESH,
    compiler_params=_SC_PARAMS,
    scratch_types=[
        pltpu.VMEM((_GCH, 64), jnp.int32),
        pltpu.VMEM((_GCH, 64), jnp.int32),
        pltpu.VMEM((_GCH, 64), jnp.float32),
        pltpu.VMEM((_GB, HH), jnp.float32),
        pltpu.VMEM((_GCH, 64), jnp.int32),
        pltpu.VMEM((_GCH, 64), jnp.int32),
        pltpu.VMEM((_GCH, 64), jnp.float32),
        pltpu.VMEM((_GB, HH), jnp.float32),
        pltpu.VMEM((_GB, HH), jnp.float32),
        pltpu.VMEM_SHARED((N_PAD, HH), jnp.float32),
        pltpu.SemaphoreType.DMA,
        pltpu.SemaphoreType.DMA,
        pltpu.SemaphoreType.DMA,
    ],
)
def _gsa_kernel(row_hbm, col_hbm, mask_hbm, xw_hbm, out_hbm,
                ridx0, cidx0, mch0, rows0, ridx1, cidx1, mch1, rows1,
                msg, accum, sem0, sem1, sem2):
    c = lax.axis_index("c")
    s = lax.axis_index("s")
    ebase = s * _GROWS
    off = c * N_PAD

    _CCHUNKS = [_GB] * (TILE_ROWS // _GB) + [TILE_ROWS % _GB]

    def _fire(ci, ridx, cidx, mch, rows, sem):
        rb = ebase + ci * _GCH
        pltpu.sync_copy(row_hbm.at[pl.ds(rb, _GCH)], ridx)
        pltpu.sync_copy(col_hbm.at[pl.ds(rb, _GCH)], cidx)
        pltpu.sync_copy(mask_hbm.at[pl.ds(rb, _GCH)], mch)
        for i in range(_GCH):
            for k in range(4):
                ridx[i, pl.ds(k * 16, 16)] = ridx[i, pl.ds(k * 16, 16)] + off
        for j in range(_GCH):
            pltpu.async_copy(xw_hbm.at[ridx.at[j]],
                             rows.at[pl.ds(j * 64, 64)], sem)

    def _wait_gathers(ridx, rows, sem):
        for j in range(_GCH):
            pltpu.make_async_copy(xw_hbm.at[ridx.at[j]],
                                  rows.at[pl.ds(j * 64, 64)], sem).wait()

    def _process(cidx, mch, rows):
        @plsc.parallel_loop(0, _GB // 16, 1)
        def _scale(g):
            m = mch[g // 4, pl.ds((g % 4) * 16, 16)]
            for j in range(16):
                e = g * 16 + j
                mj = m[j]
                msg[e, pl.ds(0, 16)] = rows[e, pl.ds(0, 16)] * mj
                msg[e, pl.ds(16, 16)] = rows[e, pl.ds(16, 16)] * mj

        descs = [pltpu.async_copy(msg.at[pl.ds(j * 64, 64)],
                                  accum.at[cidx.at[j]], sem2, add=True)
                 for j in range(_GCH)]
        for d in descs:
            d.wait()

    _fire(0, ridx0, cidx0, mch0, rows0, sem0)

    def zrow(i, carry):
        msg[i, pl.ds(0, 16)] = jnp.zeros((16,), jnp.float32)
        msg[i, pl.ds(16, 16)] = jnp.zeros((16,), jnp.float32)
        return carry

    lax.fori_loop(0, _GB, zrow, 0)
    zbase = s * TILE_ROWS
    for sz in _CCHUNKS:
        pltpu.sync_copy(msg.at[pl.ds(0, sz)], accum.at[pl.ds(zbase, sz)])
        zbase = zbase + sz
    plsc.subcore_barrier()

    def body(ci, carry):
        _fire(2 * ci + 1, ridx1, cidx1, mch1, rows1, sem1)
        _wait_gathers(ridx0, rows0, sem0)
        _process(cidx0, mch0, rows0)
        _fire(2 * ci + 2, ridx0, cidx0, mch0, rows0, sem0)
        _wait_gathers(ridx1, rows1, sem1)
        _process(cidx1, mch1, rows1)
        return carry

    lax.fori_loop(0, _GCHUNKS // 2, body, 0)
    _wait_gathers(ridx0, rows0, sem0)
    plsc.subcore_barrier()

    rb = s * TILE_ROWS
    for sz in _CCHUNKS:
        pltpu.sync_copy(accum.at[pl.ds(rb, sz)], msg.at[pl.ds(0, sz)])
        pltpu.sync_copy(msg.at[pl.ds(0, sz)], out_hbm.at[pl.ds(off + rb, sz)])
        rb = rb + sz


def kernel(x, edge_index, W_in, b_in, W1, b1, W2, b2,
           Wg0, bg0, Wg1, bg1, Wg2, bg2):
    x_pad = jnp.pad(x, ((0, N_PAD - N), (0, 0)))
    row = edge_index[0]
    col = edge_index[1]
    npad = ER_P * 64 - E
    spread = (jnp.arange(npad, dtype=jnp.int32) * 61) % N
    row2d = jnp.concatenate([row, spread]).reshape(ER_P, 64)
    col2d = jnp.concatenate([col, spread]).reshape(ER_P, 64)

    emb, A, B, xw0 = _prep_call(x_pad, W_in, b_in.reshape(1, H),
                                W1, b1.reshape(1, H), Wg0)
    mask2d = _mask_kernel(row2d, col2d, A, B, W2.reshape(H),
                          jnp.broadcast_to(b2, (16,)))
    mask2d = jnp.concatenate(
        [mask2d[:E // 64], jnp.zeros((ER_P - E // 64, 64), jnp.float32)])

    S1 = _gsa_kernel(row2d, col2d, mask2d, xw0.reshape(2 * N_PAD, HH))
    acc1, xw1 = _mid_call(S1.reshape(2, N_PAD, HH), bg0.reshape(1, H),
                          Wg1, emb)
    S2 = _gsa_kernel(row2d, col2d, mask2d, xw1.reshape(2 * N_PAD, HH))
    acc2, xw2 = _mid_call(S2.reshape(2, N_PAD, HH), bg1.reshape(1, H),
                          Wg2, acc1)
    S3 = _gsa_kernel(row2d, col2d, mask2d, xw2.reshape(2 * N_PAD, HH))
    out = _final_call(S3.reshape(2, N_PAD, HH), bg2.reshape(1, H), acc2)
    return out[:N]
